# baseline (device time: 30871 ns/iter reference)
import jax
import jax.numpy as jnp
from jax import lax
from jax.experimental import pallas as pl
from jax.experimental.pallas import tpu as pltpu

N_DEV = 4
N_STREAMS = 4
ORDER = (0, 2, 1, 3)


def kernel(x, W1, W2):
    m, k = x.shape
    d = W1.shape[1]
    n = W2.shape[1]
    mc = m // N_DEV
    qh = mc // N_STREAMS
    bf16 = jnp.bfloat16
    f32 = jnp.float32

    def body(x_hbm, w1_ref, w2_ref, out_hbm,
             xv_ref, h_ref, comm_ref, ag_ref,
             x_sems, out_sems,
             rs_send, rs_recv, ag_send, ag_recv):
        p = lax.axis_index("i")
        left = lax.rem(p + N_DEV - 1, N_DEV)
        right = lax.rem(p + 1, N_DEV)

        def mod4(v):
            return lax.rem(v + 4 * N_DEV, N_DEV)

        def is_r(st):
            return st < 2

        def row_start(c, st):
            return c * mc + st * qh

        def h_q(c, st):
            return h_ref[pl.ds(row_start(c, st), qh), :]

        def nbr(st):
            return right if is_r(st) else left

        def rs_id(st, s):
            return mod4(p - s - 1) if is_r(st) else mod4(p + s + 1)

        def ag_id(st, t):
            return mod4(p - t) if is_r(st) else mod4(p + t)

        def make(src_ref, buf, st, slot_dst, send_sems, recv_sems, hop):
            return pltpu.make_async_remote_copy(
                src_ref=src_ref,
                dst_ref=buf.at[st, slot_dst],
                send_sem=send_sems.at[st, hop],
                recv_sem=recv_sems.at[st, hop],
                device_id=(nbr(st),),
                device_id_type=pl.DeviceIdType.MESH,
            )

        barrier = pltpu.get_barrier_semaphore()
        for b in (left, right):
            pl.semaphore_signal(barrier, inc=1, device_id=(b,),
                                device_id_type=pl.DeviceIdType.MESH)

        x_copies = []
        for i, st in enumerate(ORDER):
            start = row_start(p, st)
            cp = pltpu.make_async_copy(
                x_hbm.at[pl.ds(start, qh)], xv_ref.at[pl.ds(start, qh)],
                x_sems.at[i])
            cp.start()
            x_copies.append(cp)
        for i, c in enumerate((p + 3, p + 1, p + 2)):
            start = mod4(c) * mc
            cp = pltpu.make_async_copy(
                x_hbm.at[pl.ds(start, mc)], xv_ref.at[pl.ds(start, mc)],
                x_sems.at[4 + i])
            cp.start()
            x_copies.append(cp)

        w1b = w1_ref[...].astype(bf16)

        def gemm1_q(st):
            start = row_start(p, st)
            h_ref[pl.ds(start, qh), :] = jnp.dot(
                xv_ref[pl.ds(start, qh), :].astype(bf16), w1b,
                preferred_element_type=f32).astype(bf16)

        def gemm1(c):
            start = c * mc
            h_ref[pl.ds(start, mc), :] = jnp.dot(
                xv_ref[pl.ds(start, mc), :].astype(bf16), w1b,
                preferred_element_type=f32).astype(bf16)

        rs_desc = [[None] * (N_DEV - 1) for _ in range(N_STREAMS)]
        ag_desc = [[None] * (N_DEV - 1) for _ in range(N_STREAMS)]

        def start_rs0(st):
            rs_desc[st][0] = make(
                h_ref.at[pl.ds(row_start(p, st), qh)],
                comm_ref, st, 0, rs_send, rs_recv, 0)
            rs_desc[st][0].start()

        x_copies[0].wait()
        gemm1_q(ORDER[0])
        x_copies[1].wait()
        gemm1_q(ORDER[1])
        pl.semaphore_wait(barrier, 2)
        start_rs0(ORDER[0])
        start_rs0(ORDER[1])
        x_copies[2].wait()
        gemm1_q(ORDER[2])
        x_copies[3].wait()
        gemm1_q(ORDER[3])
        start_rs0(ORDER[2])
        start_rs0(ORDER[3])

        x_copies[4].wait()
        gemm1(mod4(p + 3))
        x_copies[5].wait()
        gemm1(mod4(p + 1))
        w2b = w2_ref[...].astype(bf16)

        for s in range(N_DEV - 1):
            for st in ORDER:
                rs_desc[st][s].wait_recv()
                acc = comm_ref[st, s] + h_q(rs_id(st, s), st)
                if s < N_DEV - 2:
                    comm_ref[st, s, :, :] = acc
                    rs_desc[st][s + 1] = make(
                        comm_ref.at[st, s], comm_ref, st, s + 1,
                        rs_send, rs_recv, s + 1)
                    rs_desc[st][s + 1].start()
                else:
                    ag_ref[st, 3, :, :] = acc
                    ag_desc[st][0] = make(
                        ag_ref.at[st, 3], ag_ref, st, 0,
                        ag_send, ag_recv, 0)
                    ag_desc[st][0].start()
            if s == 0:
                x_copies[6].wait()
                gemm1(mod4(p + 2))

        out_copies = []

        def gemm2(src_val, c, st):
            start = row_start(c, st)
            h_ref[pl.ds(start, qh), :] = jnp.dot(
                src_val, w2b, preferred_element_type=f32).astype(bf16)
            cp = pltpu.make_async_copy(
                h_ref.at[pl.ds(start, qh)], out_hbm.at[pl.ds(start, qh)],
                out_sems.at[len(out_copies)])
            cp.start()
            out_copies.append(cp)

        own = {R_st: None for R_st in range(N_STREAMS)}
        for st in ORDER:
            own_c = mod4(p + 1) if is_r(st) else mod4(p - 1)
            gemm2(ag_ref[st, 3], own_c, st)

        for t in range(N_DEV - 1):
            if t < N_DEV - 2:
                for st in ORDER:
                    ag_desc[st][t].wait_recv()
                    ag_desc[st][t + 1] = make(
                        ag_ref.at[st, t], ag_ref, st, t + 1,
                        ag_send, ag_recv, t + 1)
                    ag_desc[st][t + 1].start()
                for st in ORDER:
                    gemm2(ag_ref[st, t], ag_id(st, t), st)
            else:
                for st in ORDER:
                    ag_desc[st][t].wait_recv()
                    gemm2(ag_ref[st, t], ag_id(st, t), st)

        for cp in out_copies:
            cp.wait()
        for st in range(N_STREAMS):
            for s in range(N_DEV - 1):
                rs_desc[st][s].wait_send()
                ag_desc[st][s].wait_send()

    return pl.pallas_call(
        body,
        out_shape=jax.ShapeDtypeStruct((m, n), bf16),
        in_specs=[
            pl.BlockSpec(memory_space=pl.ANY),
            pl.BlockSpec(memory_space=pltpu.VMEM),
            pl.BlockSpec(memory_space=pltpu.VMEM),
        ],
        out_specs=pl.BlockSpec(memory_space=pl.ANY),
        scratch_shapes=[
            pltpu.VMEM((m, k), jnp.float32),
            pltpu.VMEM((m, d), bf16),
            pltpu.VMEM((N_STREAMS, N_DEV - 1, qh, d), bf16),
            pltpu.VMEM((N_STREAMS, N_DEV, qh, d), bf16),
            pltpu.SemaphoreType.DMA((7,)),
            pltpu.SemaphoreType.DMA((4 * N_DEV,)),
            pltpu.SemaphoreType.DMA((N_STREAMS, N_DEV - 1)),
            pltpu.SemaphoreType.DMA((N_STREAMS, N_DEV - 1)),
            pltpu.SemaphoreType.DMA((N_STREAMS, N_DEV - 1)),
            pltpu.SemaphoreType.DMA((N_STREAMS, N_DEV - 1)),
        ],
        compiler_params=pltpu.CompilerParams(collective_id=0),
    )(x, W1, W2)


# device time: 29175 ns/iter; 1.0581x vs baseline; 1.0581x over previous
import jax
import jax.numpy as jnp
from jax import lax
from jax.experimental import pallas as pl
from jax.experimental.pallas import tpu as pltpu

N_DEV = 4
N_STREAMS = 4
ORDER = (0, 2, 1, 3)


def kernel(x, W1, W2):
    m, _ = x.shape
    d = W1.shape[1]
    n = W2.shape[1]
    mc = m // N_DEV
    qh = mc // N_STREAMS
    bf16 = jnp.bfloat16
    f32 = jnp.float32

    def body(x_ref, w1_ref, w2_ref, out_ref,
             h_ref, comm_ref, ag_ref,
             rs_send, rs_recv, ag_send, ag_recv):
        p = lax.axis_index("i")
        left = lax.rem(p + N_DEV - 1, N_DEV)
        right = lax.rem(p + 1, N_DEV)

        def mod4(v):
            return lax.rem(v + 4 * N_DEV, N_DEV)

        def is_r(st):
            return st < 2

        def row_start(c, st):
            return c * mc + st * qh

        def h_q(c, st):
            return h_ref[pl.ds(row_start(c, st), qh), :]

        def nbr(st):
            return right if is_r(st) else left

        def rs_id(st, s):
            return mod4(p - s - 1) if is_r(st) else mod4(p + s + 1)

        def ag_id(st, t):
            return mod4(p - t) if is_r(st) else mod4(p + t)

        def make(src_ref, buf, st, slot_dst, send_sems, recv_sems, hop):
            return pltpu.make_async_remote_copy(
                src_ref=src_ref,
                dst_ref=buf.at[st, slot_dst],
                send_sem=send_sems.at[st, hop],
                recv_sem=recv_sems.at[st, hop],
                device_id=(nbr(st),),
                device_id_type=pl.DeviceIdType.MESH,
            )

        barrier = pltpu.get_barrier_semaphore()
        for b in (left, right):
            pl.semaphore_signal(barrier, inc=1, device_id=(b,),
                                device_id_type=pl.DeviceIdType.MESH)

        def gemm1_q(st):
            start = row_start(p, st)
            h_ref[pl.ds(start, qh), :] = jnp.dot(
                x_ref[pl.ds(start, qh), :], w1_ref[...],
                preferred_element_type=f32).astype(bf16)

        def gemm1(c):
            start = c * mc
            h_ref[pl.ds(start, mc), :] = jnp.dot(
                x_ref[pl.ds(start, mc), :], w1_ref[...],
                preferred_element_type=f32).astype(bf16)

        rs_desc = [[None] * (N_DEV - 1) for _ in range(N_STREAMS)]
        ag_desc = [[None] * (N_DEV - 1) for _ in range(N_STREAMS)]

        def start_rs0(st):
            rs_desc[st][0] = make(
                h_ref.at[pl.ds(row_start(p, st), qh)],
                comm_ref, st, 0, rs_send, rs_recv, 0)
            rs_desc[st][0].start()

        gemm1_q(0)
        gemm1_q(2)
        pl.semaphore_wait(barrier, 2)
        start_rs0(0)
        start_rs0(2)
        gemm1_q(1)
        gemm1_q(3)
        start_rs0(1)
        start_rs0(3)

        gemm1(mod4(p + 3))
        gemm1(mod4(p + 1))

        for s in range(N_DEV - 1):
            for st in ORDER:
                rs_desc[st][s].wait_recv()
                acc = comm_ref[st, s] + h_q(rs_id(st, s), st)
                if s < N_DEV - 2:
                    comm_ref[st, s, :, :] = acc
                    rs_desc[st][s + 1] = make(
                        comm_ref.at[st, s], comm_ref, st, s + 1,
                        rs_send, rs_recv, s + 1)
                    rs_desc[st][s + 1].start()
                else:
                    ag_ref[st, 3, :, :] = acc
                    ag_desc[st][0] = make(
                        ag_ref.at[st, 3], ag_ref, st, 0,
                        ag_send, ag_recv, 0)
                    ag_desc[st][0].start()
            if s == 0:
                gemm1(mod4(p + 2))

        def gemm2(src_val, c, st):
            out_ref[pl.ds(row_start(c, st), qh), :] = jnp.dot(
                src_val, w2_ref[...], preferred_element_type=f32
            ).astype(bf16)

        for st in ORDER:
            own_c = mod4(p + 1) if is_r(st) else mod4(p - 1)
            gemm2(ag_ref[st, 3], own_c, st)

        for t in range(N_DEV - 1):
            if t < N_DEV - 2:
                for st in ORDER:
                    ag_desc[st][t].wait_recv()
                    ag_desc[st][t + 1] = make(
                        ag_ref.at[st, t], ag_ref, st, t + 1,
                        ag_send, ag_recv, t + 1)
                    ag_desc[st][t + 1].start()
                for st in ORDER:
                    gemm2(ag_ref[st, t], ag_id(st, t), st)
            else:
                for st in ORDER:
                    ag_desc[st][t].wait_recv()
                    gemm2(ag_ref[st, t], ag_id(st, t), st)

        for st in range(N_STREAMS):
            for s in range(N_DEV - 1):
                rs_desc[st][s].wait_send()
                ag_desc[st][s].wait_send()

    call = pl.pallas_call(
        body,
        out_shape=jax.ShapeDtypeStruct((m, n), bf16),
        in_specs=[pl.BlockSpec(memory_space=pltpu.VMEM)] * 3,
        out_specs=pl.BlockSpec(memory_space=pltpu.VMEM),
        scratch_shapes=[
            pltpu.VMEM((m, d), bf16),
            pltpu.VMEM((N_STREAMS, N_DEV - 1, qh, d), bf16),
            pltpu.VMEM((N_STREAMS, N_DEV, qh, d), bf16),
            pltpu.SemaphoreType.DMA((N_STREAMS, N_DEV - 1)),
            pltpu.SemaphoreType.DMA((N_STREAMS, N_DEV - 1)),
            pltpu.SemaphoreType.DMA((N_STREAMS, N_DEV - 1)),
            pltpu.SemaphoreType.DMA((N_STREAMS, N_DEV - 1)),
        ],
        compiler_params=pltpu.CompilerParams(collective_id=0),
    )
    return call(x.astype(bf16), W1.astype(bf16), W2.astype(bf16))
